# baseline (speedup 1.0000x reference)
import functools

import jax
import jax.numpy as jnp
from jax import lax
from jax.experimental import pallas as pl
from jax.experimental.pallas import tpu as pltpu
from jax.experimental.pallas import tpu_sc as plsc

_VOCAB = 1000000
_D = 32
_H = 64
_C = 3
_B = 16384
_L = 50

_NC = 2
_NS = 16
_NW = _NC * _NS

_CB = 32
_CHUNK = 80
_IDX_PER_STEP = _CB * _L
_NCHUNK = _IDX_PER_STEP // _CHUNK
_ROWS_PER_W = _B // _NW
_NSTEP = _ROWS_PER_W // _CB


def _pool_body(emb_hbm, ids_hbm, out_hbm,
               idx0, idx1, rows0, rows1, acc_v, sem0, sem1):
    wid = lax.axis_index("s") * _NC + lax.axis_index("c")
    base_row = wid * _ROWS_PER_W
    idx_bufs = (idx0, idx1)
    rows_bufs = (rows0, rows1)
    sems = (sem0, sem1)

    def stage_and_fire(s, p):
        row0 = base_row + s * _CB
        pltpu.sync_copy(ids_hbm.at[pl.ds(row0 * _L, _IDX_PER_STEP)],
                        idx_bufs[p])
        for c in range(_NCHUNK):
            off = c * _CHUNK
            pltpu.async_copy(
                emb_hbm.at[idx_bufs[p].at[pl.ds(off, _CHUNK)]],
                rows_bufs[p].at[pl.ds(off, _CHUNK)],
                sems[p],
            )

    inv = jnp.float32(1.0 / _L)
    stage_and_fire(0, 0)
    for s in range(_NSTEP):
        p = s % 2
        if s + 1 < _NSTEP:
            stage_and_fire(s + 1, (s + 1) % 2)
        pltpu.make_async_copy(
            emb_hbm.at[pl.ds(0, _IDX_PER_STEP)], rows_bufs[p], sems[p]
        ).wait()

        rows_v = rows_bufs[p]

        def pool_one(i, carry):
            j0 = i * _L

            def add_tok(l, acc):
                j = j0 + l
                return (acc[0] + rows_v[j, pl.ds(0, 16)],
                        acc[1] + rows_v[j, pl.ds(16, 16)])

            a0, a1 = lax.fori_loop(
                0, _L, add_tok,
                (jnp.zeros((16,), jnp.float32), jnp.zeros((16,), jnp.float32)),
                unroll=True,
            )
            acc_v[i, pl.ds(0, 16)] = a0 * inv
            acc_v[i, pl.ds(16, 16)] = a1 * inv
            return carry

        lax.fori_loop(0, _CB, pool_one, 0)
        pltpu.sync_copy(acc_v, out_hbm.at[pl.ds(base_row + s * _CB, _CB)])


@jax.jit
def _pool(ids_flat, emb):
    mesh = plsc.VectorSubcoreMesh(core_axis_name="c", subcore_axis_name="s")
    return pl.kernel(
        _pool_body,
        out_type=jax.ShapeDtypeStruct((_B, _D), jnp.float32),
        mesh=mesh,
        scratch_types=[
            pltpu.VMEM((_IDX_PER_STEP,), jnp.int32),
            pltpu.VMEM((_IDX_PER_STEP,), jnp.int32),
            pltpu.VMEM((_IDX_PER_STEP, _D), jnp.float32),
            pltpu.VMEM((_IDX_PER_STEP, _D), jnp.float32),
            pltpu.VMEM((_CB, _D), jnp.float32),
            pltpu.SemaphoreType.DMA,
            pltpu.SemaphoreType.DMA,
        ],
        compiler_params=pltpu.CompilerParams(use_tc_tiling_on_sc=False),
    )(emb, ids_flat)


_RK = 6
_RCW = _RK * 128
_NFULL = _VOCAB // 128
_RNCHUNK = _NFULL // _RK
_TAIL_COL = _NFULL * 128
_TAIL_W = _VOCAB - _TAIL_COL


def _relayout_body(embt_hbm, tail_hbm, out_hbm, s0, s1, s2, s3, out_v, sem):
    wid = lax.axis_index("s") * _NC + lax.axis_index("c")
    slabs = (s0, s1, s2, s3)
    lane32 = lax.iota(jnp.int32, 16) * _D

    nt = jnp.where(wid < _RNCHUNK % _NW, _RNCHUNK // _NW + 1, _RNCHUNK // _NW)

    def step(t, carry):
        c = wid + t * _NW
        col0 = pl.multiple_of(c * _RCW, 128)
        for jb in range(4):
            pltpu.async_copy(
                embt_hbm.at[pl.ds(jb * 8, 8), pl.ds(col0, _RCW)],
                slabs[jb], sem)
        for jb in range(4):
            pltpu.make_async_copy(
                embt_hbm.at[pl.ds(jb * 8, 8), pl.ds(0, _RCW)],
                slabs[jb], sem).wait()

        def tgroup(cg, carry2):
            for jb in range(4):
                for jr in range(8):
                    d = jb * 8 + jr
                    vals = slabs[jb][jr, pl.ds(cg * 16, 16)]
                    idx = lane32 + (cg * 16 * _D + d)
                    plsc.store_scatter(out_v, [idx], vals)
            return carry2

        lax.fori_loop(0, _RCW // 16, tgroup, 0)
        pltpu.sync_copy(out_v, out_hbm.at[pl.ds(col0 * _D, _RCW * _D)])
        return carry

    lax.fori_loop(0, nt, step, 0)

    @pl.when(wid == _NW - 1)
    def _tail():
        pltpu.sync_copy(tail_hbm, out_v.at[pl.ds(0, _TAIL_W * _D)])
        pltpu.sync_copy(out_v.at[pl.ds(0, _TAIL_W * _D)],
                        out_hbm.at[pl.ds(_TAIL_COL * _D, _TAIL_W * _D)])


@jax.jit
def _relayout(emb):
    mesh = plsc.VectorSubcoreMesh(core_axis_name="c", subcore_axis_name="s")
    out_flat = pl.kernel(
        _relayout_body,
        out_type=jax.ShapeDtypeStruct((_VOCAB * _D,), jnp.float32),
        mesh=mesh,
        scratch_types=[
            pltpu.VMEM((8, _RCW), jnp.float32),
            pltpu.VMEM((8, _RCW), jnp.float32),
            pltpu.VMEM((8, _RCW), jnp.float32),
            pltpu.VMEM((8, _RCW), jnp.float32),
            pltpu.VMEM((_RCW * _D,), jnp.float32),
            pltpu.SemaphoreType.DMA,
        ],
        compiler_params=pltpu.CompilerParams(use_tc_tiling_on_sc=True,
                                             needs_layout_passes=False),
    )(emb.T, emb[_TAIL_COL:].reshape(-1))
    return out_flat.reshape(_VOCAB, _D)


def _mlp_body(x_ref, w1_ref, b1_ref, w2_ref, b2_ref, o_ref):
    x = x_ref[...]
    h = jnp.dot(x, w1_ref[...], preferred_element_type=jnp.float32)
    h = jnp.maximum(h + b1_ref[...], 0.0)
    o_ref[...] = (
        jnp.dot(h, w2_ref[...], preferred_element_type=jnp.float32)
        + b2_ref[...]
    )


@jax.jit
def _mlp(x, W1, b1, W2, b2):
    blk = 2048
    grid = _B // blk
    return pl.pallas_call(
        _mlp_body,
        grid=(grid,),
        in_specs=[
            pl.BlockSpec((blk, _D), lambda i: (i, 0)),
            pl.BlockSpec((_D, _H), lambda i: (0, 0)),
            pl.BlockSpec((1, _H), lambda i: (0, 0)),
            pl.BlockSpec((_H, _C), lambda i: (0, 0)),
            pl.BlockSpec((1, _C), lambda i: (0, 0)),
        ],
        out_specs=pl.BlockSpec((blk, _C), lambda i: (i, 0)),
        out_shape=jax.ShapeDtypeStruct((_B, _C), jnp.float32),
    )(x, W1, b1.reshape(1, _H), W2, b2.reshape(1, _C))


def kernel(ids, emb, W1, b1, W2, b2):
    ids_flat = ids.reshape(-1).astype(jnp.int32)
    emb_lin = _relayout(emb)
    pooled = _pool(ids_flat, emb_lin)
    return _mlp(pooled, W1, b1, W2, b2)


# speedup vs baseline: 2.0477x; 2.0477x over previous
import functools

import jax
import jax.numpy as jnp
from jax import lax
from jax.experimental import pallas as pl
from jax.experimental.pallas import tpu as pltpu
from jax.experimental.pallas import tpu_sc as plsc

_VOCAB = 1000000
_D = 32
_H = 64
_C = 3
_B = 16384
_L = 50

_NC = 2
_NS = 16
_NW = _NC * _NS

_CB = 32
_CHUNK = 80
_IDX_PER_STEP = _CB * _L
_NCHUNK = _IDX_PER_STEP // _CHUNK
_ROWS_PER_W = _B // _NW
_NSTEP = _ROWS_PER_W // _CB


def _pool_body(emb_hbm, ids_hbm, out_hbm,
               idx0, idx1, rows0, rows1, acc_v, sem0, sem1):
    wid = lax.axis_index("s") * _NC + lax.axis_index("c")
    base_row = wid * _ROWS_PER_W
    idx_bufs = (idx0, idx1)
    rows_bufs = (rows0, rows1)
    sems = (sem0, sem1)

    def stage_and_fire(s, p):
        row0 = base_row + s * _CB
        pltpu.sync_copy(ids_hbm.at[pl.ds(row0 * _L, _IDX_PER_STEP)],
                        idx_bufs[p])
        for c in range(_NCHUNK):
            off = c * _CHUNK
            pltpu.async_copy(
                emb_hbm.at[idx_bufs[p].at[pl.ds(off, _CHUNK)]],
                rows_bufs[p].at[pl.ds(off, _CHUNK)],
                sems[p],
            )

    inv = jnp.float32(1.0 / _L)
    stage_and_fire(0, 0)
    for s in range(_NSTEP):
        p = s % 2
        if s + 1 < _NSTEP:
            stage_and_fire(s + 1, (s + 1) % 2)
        pltpu.make_async_copy(
            emb_hbm.at[pl.ds(0, _IDX_PER_STEP)], rows_bufs[p], sems[p]
        ).wait()

        rows_v = rows_bufs[p]

        def pool_one(i, carry):
            j0 = i * _L

            def add_tok(l, acc):
                j = j0 + l
                return (acc[0] + rows_v[j, pl.ds(0, 16)],
                        acc[1] + rows_v[j, pl.ds(16, 16)])

            a0, a1 = lax.fori_loop(
                0, _L, add_tok,
                (jnp.zeros((16,), jnp.float32), jnp.zeros((16,), jnp.float32)),
                unroll=True,
            )
            acc_v[i, pl.ds(0, 16)] = a0 * inv
            acc_v[i, pl.ds(16, 16)] = a1 * inv
            return carry

        lax.fori_loop(0, _CB, pool_one, 0)
        pltpu.sync_copy(acc_v, out_hbm.at[pl.ds(base_row + s * _CB, _CB)])


@jax.jit
def _pool(ids_flat, emb):
    mesh = plsc.VectorSubcoreMesh(core_axis_name="c", subcore_axis_name="s")
    return pl.kernel(
        _pool_body,
        out_type=jax.ShapeDtypeStruct((_B, _D), jnp.float32),
        mesh=mesh,
        scratch_types=[
            pltpu.VMEM((_IDX_PER_STEP,), jnp.int32),
            pltpu.VMEM((_IDX_PER_STEP,), jnp.int32),
            pltpu.VMEM((_IDX_PER_STEP, _D), jnp.float32),
            pltpu.VMEM((_IDX_PER_STEP, _D), jnp.float32),
            pltpu.VMEM((_CB, _D), jnp.float32),
            pltpu.SemaphoreType.DMA,
            pltpu.SemaphoreType.DMA,
        ],
        compiler_params=pltpu.CompilerParams(use_tc_tiling_on_sc=False),
    )(emb, ids_flat)


_RK = 6
_RCW = _RK * 128
_NFULL = _VOCAB // 128
_RNCHUNK = _NFULL // _RK
_TAIL_COL = _NFULL * 128
_TAIL_W = _VOCAB - _TAIL_COL


def _relayout_body(embt_hbm, tail_hbm, out_hbm,
                   sl0, sl1, ov0, ov1, sem0, sem1):
    wid = lax.axis_index("s") * _NC + lax.axis_index("c")
    lane = lax.iota(jnp.int32, 16)
    lane32 = lane * _D

    nchunk_w = jnp.where(wid < _RNCHUNK % _NW,
                         _RNCHUNK // _NW + 1, _RNCHUNK // _NW)

    def fire(j, sl, sem):
        col0 = pl.multiple_of((wid + j * _NW) * _RCW, 128)
        for jb in range(4):
            pltpu.async_copy(
                embt_hbm.at[pl.ds(jb * 8, 8), pl.ds(col0, _RCW)],
                sl.at[pl.ds(jb * 8, 8), pl.ds(0, _RCW)], sem)

    def process(j, sl, ov, sem):
        pltpu.make_async_copy(
            embt_hbm.at[pl.ds(0, 32), pl.ds(0, _RCW)], sl, sem).wait()

        def rblock(rb, carry):
            r0 = rb * 16
            cvec = lane + r0
            sbase = r0 * _D
            for db in range(2):
                for k in range(16):
                    perm = lax.rem(lane + k, 16)
                    dvec = perm + db * 16
                    vals = plsc.load_gather(sl, [dvec, cvec])
                    sidx = lane32 + (perm + (sbase + db * 16))
                    plsc.store_scatter(ov, [sidx], vals)
            return carry

        lax.fori_loop(0, _RCW // 16, rblock, 0)
        col0 = pl.multiple_of((wid + j * _NW) * _RCW, 128)
        pltpu.sync_copy(ov, out_hbm.at[pl.ds(col0 * _D, _RCW * _D)])

    fire(0, sl0, sem0)
    nt2 = (nchunk_w + 1) // 2

    def step(t, carry):
        j0, j1, j2 = 2 * t, 2 * t + 1, 2 * t + 2

        @pl.when(j1 < nchunk_w)
        def _():
            fire(j1, sl1, sem1)

        process(j0, sl0, ov0, sem0)

        @pl.when(j2 < nchunk_w)
        def _():
            fire(j2, sl0, sem0)

        @pl.when(j1 < nchunk_w)
        def _():
            process(j1, sl1, ov1, sem1)

        return carry

    lax.fori_loop(0, nt2, step, 0)

    @pl.when(wid == _NW - 1)
    def _tail():
        pltpu.sync_copy(tail_hbm, ov0.at[pl.ds(0, _TAIL_W * _D)])
        pltpu.sync_copy(ov0.at[pl.ds(0, _TAIL_W * _D)],
                        out_hbm.at[pl.ds(_TAIL_COL * _D, _TAIL_W * _D)])


@jax.jit
def _relayout(emb):
    mesh = plsc.VectorSubcoreMesh(core_axis_name="c", subcore_axis_name="s")
    out_flat = pl.kernel(
        _relayout_body,
        out_type=jax.ShapeDtypeStruct((_VOCAB * _D,), jnp.float32),
        mesh=mesh,
        scratch_types=[
            pltpu.VMEM((32, _RCW), jnp.float32),
            pltpu.VMEM((32, _RCW), jnp.float32),
            pltpu.VMEM((_RCW * _D,), jnp.float32),
            pltpu.VMEM((_RCW * _D,), jnp.float32),
            pltpu.SemaphoreType.DMA,
            pltpu.SemaphoreType.DMA,
        ],
        compiler_params=pltpu.CompilerParams(use_tc_tiling_on_sc=True,
                                             needs_layout_passes=False),
    )(emb.T, emb[_TAIL_COL:].reshape(-1))
    return out_flat.reshape(_VOCAB, _D)


def _mlp_body(x_ref, w1_ref, b1_ref, w2_ref, b2_ref, o_ref):
    x = x_ref[...]
    h = jnp.dot(x, w1_ref[...], preferred_element_type=jnp.float32)
    h = jnp.maximum(h + b1_ref[...], 0.0)
    o_ref[...] = (
        jnp.dot(h, w2_ref[...], preferred_element_type=jnp.float32)
        + b2_ref[...]
    )


@jax.jit
def _mlp(x, W1, b1, W2, b2):
    blk = 2048
    grid = _B // blk
    return pl.pallas_call(
        _mlp_body,
        grid=(grid,),
        in_specs=[
            pl.BlockSpec((blk, _D), lambda i: (i, 0)),
            pl.BlockSpec((_D, _H), lambda i: (0, 0)),
            pl.BlockSpec((1, _H), lambda i: (0, 0)),
            pl.BlockSpec((_H, _C), lambda i: (0, 0)),
            pl.BlockSpec((1, _C), lambda i: (0, 0)),
        ],
        out_specs=pl.BlockSpec((blk, _C), lambda i: (i, 0)),
        out_shape=jax.ShapeDtypeStruct((_B, _C), jnp.float32),
    )(x, W1, b1.reshape(1, _H), W2, b2.reshape(1, _C))


def kernel(ids, emb, W1, b1, W2, b2):
    ids_flat = ids.reshape(-1).astype(jnp.int32)
    emb_lin = _relayout(emb)
    pooled = _pool(ids_flat, emb_lin)
    return _mlp(pooled, W1, b1, W2, b2)


# speedup vs baseline: 3.7636x; 1.8380x over previous
import functools

import jax
import jax.numpy as jnp
from jax import lax
from jax.experimental import pallas as pl
from jax.experimental.pallas import tpu as pltpu
from jax.experimental.pallas import tpu_sc as plsc

_VOCAB = 1000000
_D = 32
_H = 64
_C = 3
_B = 16384
_L = 50

_NC = 2
_NS = 16
_NW = _NC * _NS

_CB = 32
_CHUNK = 80
_IDX_PER_STEP = _CB * _L
_NCHUNK = _IDX_PER_STEP // _CHUNK
_ROWS_PER_W = _B // _NW
_NSTEP = _ROWS_PER_W // _CB


def _pool_body(emb_hbm, ids_hbm, out_hbm,
               idx0, idx1, rows0, rows1, acc_v, sem0, sem1):
    wid = lax.axis_index("s") * _NC + lax.axis_index("c")
    base_row = wid * _ROWS_PER_W
    idx_bufs = (idx0, idx1)
    rows_bufs = (rows0, rows1)
    sems = (sem0, sem1)

    def stage_and_fire(s, p):
        row0 = base_row + s * _CB
        pltpu.sync_copy(ids_hbm.at[pl.ds(row0 * _L, _IDX_PER_STEP)],
                        idx_bufs[p])
        for c in range(_NCHUNK):
            off = c * _CHUNK
            pltpu.async_copy(
                emb_hbm.at[idx_bufs[p].at[pl.ds(off, _CHUNK)]],
                rows_bufs[p].at[pl.ds(off, _CHUNK)],
                sems[p],
            )

    inv = jnp.float32(1.0 / _L)
    stage_and_fire(0, 0)
    for s in range(_NSTEP):
        p = s % 2
        if s + 1 < _NSTEP:
            stage_and_fire(s + 1, (s + 1) % 2)
        pltpu.make_async_copy(
            emb_hbm.at[pl.ds(0, _IDX_PER_STEP)], rows_bufs[p], sems[p]
        ).wait()

        rows_v = rows_bufs[p]

        def pool_one(i, carry):
            j0 = i * _L

            def add_tok(l, acc):
                j = j0 + l
                return (acc[0] + rows_v[j, pl.ds(0, 16)],
                        acc[1] + rows_v[j, pl.ds(16, 16)])

            a0, a1 = lax.fori_loop(
                0, _L, add_tok,
                (jnp.zeros((16,), jnp.float32), jnp.zeros((16,), jnp.float32)),
                unroll=True,
            )
            acc_v[i, pl.ds(0, 16)] = a0 * inv
            acc_v[i, pl.ds(16, 16)] = a1 * inv
            return carry

        lax.fori_loop(0, _CB, pool_one, 0)
        pltpu.sync_copy(acc_v, out_hbm.at[pl.ds(base_row + s * _CB, _CB)])


@jax.jit
def _pool(ids_flat, emb):
    mesh = plsc.VectorSubcoreMesh(core_axis_name="c", subcore_axis_name="s")
    return pl.kernel(
        _pool_body,
        out_type=jax.ShapeDtypeStruct((_B, _D), jnp.float32),
        mesh=mesh,
        scratch_types=[
            pltpu.VMEM((_IDX_PER_STEP,), jnp.int32),
            pltpu.VMEM((_IDX_PER_STEP,), jnp.int32),
            pltpu.VMEM((_IDX_PER_STEP, _D), jnp.float32),
            pltpu.VMEM((_IDX_PER_STEP, _D), jnp.float32),
            pltpu.VMEM((_CB, _D), jnp.float32),
            pltpu.SemaphoreType.DMA,
            pltpu.SemaphoreType.DMA,
        ],
        compiler_params=pltpu.CompilerParams(use_tc_tiling_on_sc=False),
    )(emb, ids_flat)


_RK = 6
_RCW = _RK * 128
_NFULL = _VOCAB // 128
_RNCHUNK = _NFULL // _RK
_TAIL_COL = _NFULL * 128
_TAIL_W = _VOCAB - _TAIL_COL


def _relayout_body(embt_hbm, tail_hbm, out_hbm,
                   sl0, sl1, ov0, ov1, sem0, sem1):
    wid = lax.axis_index("s") * _NC + lax.axis_index("c")
    lane = lax.iota(jnp.int32, 16)
    lane32 = lane * _D

    nchunk_w = jnp.where(wid < _RNCHUNK % _NW,
                         _RNCHUNK // _NW + 1, _RNCHUNK // _NW)

    def fire(j, sl, sem):
        col0 = pl.multiple_of((wid + j * _NW) * _RCW, 128)
        for jb in range(4):
            pltpu.async_copy(
                embt_hbm.at[pl.ds(jb * 8, 8), pl.ds(col0, _RCW)],
                sl.at[pl.ds(jb * 8, 8), pl.ds(0, _RCW)], sem)

    def process(j, sl, ov, sem):
        pltpu.make_async_copy(
            embt_hbm.at[pl.ds(0, 32), pl.ds(0, _RCW)], sl, sem).wait()

        @plsc.parallel_loop(0, _RCW // 16)
        def _rblock(rb):
            r0 = rb * 16
            cvec = lane + r0
            sbase = r0 * _D
            for db in range(2):
                for g in range(0, 16, 4):
                    pairs = []
                    for k in range(g, g + 4):
                        perm = jnp.bitwise_and(lane + k, 15)
                        vals = plsc.load_gather(sl, [perm + db * 16, cvec])
                        pairs.append((lane32 + perm + (sbase + db * 16), vals))
                    for sidx, vals in pairs:
                        plsc.store_scatter(ov, [sidx], vals)

        col0 = pl.multiple_of((wid + j * _NW) * _RCW, 128)
        pltpu.sync_copy(ov, out_hbm.at[pl.ds(col0 * _D, _RCW * _D)])

    fire(0, sl0, sem0)
    nt2 = (nchunk_w + 1) // 2

    def step(t, carry):
        j0, j1, j2 = 2 * t, 2 * t + 1, 2 * t + 2

        @pl.when(j1 < nchunk_w)
        def _():
            fire(j1, sl1, sem1)

        process(j0, sl0, ov0, sem0)

        @pl.when(j2 < nchunk_w)
        def _():
            fire(j2, sl0, sem0)

        @pl.when(j1 < nchunk_w)
        def _():
            process(j1, sl1, ov1, sem1)

        return carry

    lax.fori_loop(0, nt2, step, 0)

    @pl.when(wid == _NW - 1)
    def _tail():
        pltpu.sync_copy(tail_hbm, ov0.at[pl.ds(0, _TAIL_W * _D)])
        pltpu.sync_copy(ov0.at[pl.ds(0, _TAIL_W * _D)],
                        out_hbm.at[pl.ds(_TAIL_COL * _D, _TAIL_W * _D)])


@jax.jit
def _relayout(emb):
    mesh = plsc.VectorSubcoreMesh(core_axis_name="c", subcore_axis_name="s")
    out_flat = pl.kernel(
        _relayout_body,
        out_type=jax.ShapeDtypeStruct((_VOCAB * _D,), jnp.float32),
        mesh=mesh,
        scratch_types=[
            pltpu.VMEM((32, _RCW), jnp.float32),
            pltpu.VMEM((32, _RCW), jnp.float32),
            pltpu.VMEM((_RCW * _D,), jnp.float32),
            pltpu.VMEM((_RCW * _D,), jnp.float32),
            pltpu.SemaphoreType.DMA,
            pltpu.SemaphoreType.DMA,
        ],
        compiler_params=pltpu.CompilerParams(use_tc_tiling_on_sc=True,
                                             needs_layout_passes=False,
                                             disable_bounds_checks=True),
    )(emb.T, emb[_TAIL_COL:].reshape(-1))
    return out_flat.reshape(_VOCAB, _D)


def _mlp_body(x_ref, w1_ref, b1_ref, w2_ref, b2_ref, o_ref):
    x = x_ref[...]
    h = jnp.dot(x, w1_ref[...], preferred_element_type=jnp.float32)
    h = jnp.maximum(h + b1_ref[...], 0.0)
    o_ref[...] = (
        jnp.dot(h, w2_ref[...], preferred_element_type=jnp.float32)
        + b2_ref[...]
    )


@jax.jit
def _mlp(x, W1, b1, W2, b2):
    blk = 2048
    grid = _B // blk
    return pl.pallas_call(
        _mlp_body,
        grid=(grid,),
        in_specs=[
            pl.BlockSpec((blk, _D), lambda i: (i, 0)),
            pl.BlockSpec((_D, _H), lambda i: (0, 0)),
            pl.BlockSpec((1, _H), lambda i: (0, 0)),
            pl.BlockSpec((_H, _C), lambda i: (0, 0)),
            pl.BlockSpec((1, _C), lambda i: (0, 0)),
        ],
        out_specs=pl.BlockSpec((blk, _C), lambda i: (i, 0)),
        out_shape=jax.ShapeDtypeStruct((_B, _C), jnp.float32),
    )(x, W1, b1.reshape(1, _H), W2, b2.reshape(1, _C))


def kernel(ids, emb, W1, b1, W2, b2):
    ids_flat = ids.reshape(-1).astype(jnp.int32)
    emb_lin = _relayout(emb)
    pooled = _pool(ids_flat, emb_lin)
    return _mlp(pooled, W1, b1, W2, b2)


# speedup vs baseline: 4.1004x; 1.0895x over previous
import functools

import jax
import jax.numpy as jnp
from jax import lax
from jax.experimental import pallas as pl
from jax.experimental.pallas import tpu as pltpu
from jax.experimental.pallas import tpu_sc as plsc

_VOCAB = 1000000
_D = 32
_H = 64
_C = 3
_B = 16384
_L = 50

_NC = 2
_NS = 16
_NW = _NC * _NS

_CB = 32
_CHUNK = 80
_IDX_PER_STEP = _CB * _L
_NCHUNK = _IDX_PER_STEP // _CHUNK
_ROWS_PER_W = _B // _NW
_NSTEP = _ROWS_PER_W // _CB


def _pool_body(emb_hbm, ids_hbm, out_hbm,
               idx0, idx1, rows0, rows1, acc_v, sem0, sem1):
    wid = lax.axis_index("s") * _NC + lax.axis_index("c")
    base_row = wid * _ROWS_PER_W
    idx_bufs = (idx0, idx1)
    rows_bufs = (rows0, rows1)
    sems = (sem0, sem1)

    def stage_and_fire(s, p):
        row0 = base_row + s * _CB
        pltpu.sync_copy(ids_hbm.at[pl.ds(row0 * _L, _IDX_PER_STEP)],
                        idx_bufs[p])
        for c in range(_NCHUNK):
            off = c * _CHUNK
            pltpu.async_copy(
                emb_hbm.at[idx_bufs[p].at[pl.ds(off, _CHUNK)]],
                rows_bufs[p].at[pl.ds(off, _CHUNK)],
                sems[p],
            )

    inv = jnp.float32(1.0 / _L)
    stage_and_fire(0, 0)
    for s in range(_NSTEP):
        p = s % 2
        if s + 1 < _NSTEP:
            stage_and_fire(s + 1, (s + 1) % 2)
        pltpu.make_async_copy(
            emb_hbm.at[pl.ds(0, _IDX_PER_STEP)], rows_bufs[p], sems[p]
        ).wait()

        rows_v = rows_bufs[p]

        def pool_one(i, carry):
            j0 = i * _L

            def add_tok(l, acc):
                j = j0 + l
                return (acc[0] + rows_v[j, pl.ds(0, 16)],
                        acc[1] + rows_v[j, pl.ds(16, 16)])

            a0, a1 = lax.fori_loop(
                0, _L, add_tok,
                (jnp.zeros((16,), jnp.float32), jnp.zeros((16,), jnp.float32)),
                unroll=True,
            )
            acc_v[i, pl.ds(0, 16)] = a0 * inv
            acc_v[i, pl.ds(16, 16)] = a1 * inv
            return carry

        lax.fori_loop(0, _CB, pool_one, 0)
        pltpu.sync_copy(acc_v, out_hbm.at[pl.ds(base_row + s * _CB, _CB)])


@jax.jit
def _pool(ids_flat, emb):
    mesh = plsc.VectorSubcoreMesh(core_axis_name="c", subcore_axis_name="s")
    return pl.kernel(
        _pool_body,
        out_type=jax.ShapeDtypeStruct((_B, _D), jnp.float32),
        mesh=mesh,
        scratch_types=[
            pltpu.VMEM((_IDX_PER_STEP,), jnp.int32),
            pltpu.VMEM((_IDX_PER_STEP,), jnp.int32),
            pltpu.VMEM((_IDX_PER_STEP, _D), jnp.float32),
            pltpu.VMEM((_IDX_PER_STEP, _D), jnp.float32),
            pltpu.VMEM((_CB, _D), jnp.float32),
            pltpu.SemaphoreType.DMA,
            pltpu.SemaphoreType.DMA,
        ],
        compiler_params=pltpu.CompilerParams(use_tc_tiling_on_sc=False),
    )(emb, ids_flat)


_RK = 7
_RCW = _RK * 128
_NFULL = _VOCAB // 128
_RNCHUNK = _NFULL // _RK
_TAIL_COL = _NFULL * 128
_TAIL_W = _VOCAB - _TAIL_COL


def _relayout_body(embt_hbm, tail_hbm, out_hbm,
                   sl0, sl1, ov0, ov1, sem0, sem1, semo0, semo1):
    wid = lax.axis_index("s") * _NC + lax.axis_index("c")
    lane = lax.iota(jnp.int32, 16)
    lane32 = lane * _D

    nchunk_w = jnp.where(wid < _RNCHUNK % _NW,
                         _RNCHUNK // _NW + 1, _RNCHUNK // _NW)

    def fire(j, sl, sem):
        col0 = pl.multiple_of((wid + j * _NW) * _RCW, 128)
        for jb in range(4):
            pltpu.async_copy(
                embt_hbm.at[pl.ds(jb * 8, 8), pl.ds(col0, _RCW)],
                sl.at[pl.ds(jb * 8, 8), pl.ds(0, _RCW)], sem)

    def process(j, t, sl, ov, sem, semo):
        pltpu.make_async_copy(
            embt_hbm.at[pl.ds(0, 32), pl.ds(0, _RCW)], sl, sem).wait()

        @pl.when(t > 0)
        def _():
            pltpu.make_async_copy(
                out_hbm.at[pl.ds(0, _RCW * _D)], ov, semo).wait()

        @plsc.parallel_loop(0, _RCW // 16)
        def _rblock(rb):
            r0 = rb * 16
            cvec = lane + r0
            sbase = r0 * _D
            for db in range(2):
                for g in range(0, 16, 4):
                    pairs = []
                    for k in range(g, g + 4):
                        perm = jnp.bitwise_and(lane + k, 15)
                        vals = plsc.load_gather(sl, [perm + db * 16, cvec])
                        pairs.append((lane32 + perm + (sbase + db * 16), vals))
                    for sidx, vals in pairs:
                        plsc.store_scatter(ov, [sidx], vals)

        col0 = pl.multiple_of((wid + j * _NW) * _RCW, 128)
        pltpu.async_copy(ov, out_hbm.at[pl.ds(col0 * _D, _RCW * _D)], semo)

    fire(0, sl0, sem0)
    nt2 = (nchunk_w + 1) // 2

    def step(t, carry):
        j0, j1, j2 = 2 * t, 2 * t + 1, 2 * t + 2

        @pl.when(j1 < nchunk_w)
        def _():
            fire(j1, sl1, sem1)

        process(j0, t, sl0, ov0, sem0, semo0)

        @pl.when(j2 < nchunk_w)
        def _():
            fire(j2, sl0, sem0)

        @pl.when(j1 < nchunk_w)
        def _():
            process(j1, t, sl1, ov1, sem1, semo1)

        return carry

    lax.fori_loop(0, nt2, step, 0)
    pltpu.make_async_copy(
        out_hbm.at[pl.ds(0, _RCW * _D)], ov0, semo0).wait()
    pltpu.make_async_copy(
        out_hbm.at[pl.ds(0, _RCW * _D)], ov1, semo1).wait()

    @pl.when(wid == _NW - 1)
    def _tail():
        pltpu.sync_copy(tail_hbm, ov0.at[pl.ds(0, _TAIL_W * _D)])
        pltpu.sync_copy(ov0.at[pl.ds(0, _TAIL_W * _D)],
                        out_hbm.at[pl.ds(_TAIL_COL * _D, _TAIL_W * _D)])


@jax.jit
def _relayout(emb):
    mesh = plsc.VectorSubcoreMesh(core_axis_name="c", subcore_axis_name="s")
    out_flat = pl.kernel(
        _relayout_body,
        out_type=jax.ShapeDtypeStruct((_VOCAB * _D,), jnp.float32),
        mesh=mesh,
        scratch_types=[
            pltpu.VMEM((32, _RCW), jnp.float32),
            pltpu.VMEM((32, _RCW), jnp.float32),
            pltpu.VMEM((_RCW * _D,), jnp.float32),
            pltpu.VMEM((_RCW * _D,), jnp.float32),
            pltpu.SemaphoreType.DMA,
            pltpu.SemaphoreType.DMA,
            pltpu.SemaphoreType.DMA,
            pltpu.SemaphoreType.DMA,
        ],
        compiler_params=pltpu.CompilerParams(use_tc_tiling_on_sc=True,
                                             needs_layout_passes=False,
                                             disable_bounds_checks=True),
    )(emb.T, emb[_TAIL_COL:].reshape(-1))
    return out_flat.reshape(_VOCAB, _D)


def _mlp_body(x_ref, w1_ref, b1_ref, w2_ref, b2_ref, o_ref):
    x = x_ref[...]
    h = jnp.dot(x, w1_ref[...], preferred_element_type=jnp.float32)
    h = jnp.maximum(h + b1_ref[...], 0.0)
    o_ref[...] = (
        jnp.dot(h, w2_ref[...], preferred_element_type=jnp.float32)
        + b2_ref[...]
    )


@jax.jit
def _mlp(x, W1, b1, W2, b2):
    blk = 2048
    grid = _B // blk
    return pl.pallas_call(
        _mlp_body,
        grid=(grid,),
        in_specs=[
            pl.BlockSpec((blk, _D), lambda i: (i, 0)),
            pl.BlockSpec((_D, _H), lambda i: (0, 0)),
            pl.BlockSpec((1, _H), lambda i: (0, 0)),
            pl.BlockSpec((_H, _C), lambda i: (0, 0)),
            pl.BlockSpec((1, _C), lambda i: (0, 0)),
        ],
        out_specs=pl.BlockSpec((blk, _C), lambda i: (i, 0)),
        out_shape=jax.ShapeDtypeStruct((_B, _C), jnp.float32),
    )(x, W1, b1.reshape(1, _H), W2, b2.reshape(1, _C))


def kernel(ids, emb, W1, b1, W2, b2):
    ids_flat = ids.reshape(-1).astype(jnp.int32)
    emb_lin = _relayout(emb)
    pooled = _pool(ids_flat, emb_lin)
    return _mlp(pooled, W1, b1, W2, b2)


# speedup vs baseline: 4.2961x; 1.0477x over previous
import functools

import jax
import jax.numpy as jnp
from jax import lax
from jax.experimental import pallas as pl
from jax.experimental.pallas import tpu as pltpu
from jax.experimental.pallas import tpu_sc as plsc

_VOCAB = 1000000
_D = 32
_H = 64
_C = 3
_B = 16384
_L = 50

_NC = 2
_NS = 16
_NW = _NC * _NS

_CB = 32
_CHUNK = 80
_IDX_PER_STEP = _CB * _L
_NCHUNK = _IDX_PER_STEP // _CHUNK
_ROWS_PER_W = _B // _NW
_NSTEP = _ROWS_PER_W // _CB


def _pool_body(emb_hbm, ids_hbm, out_hbm,
               idx0, idx1, rows0, rows1, acc_v, sem0, sem1):
    wid = lax.axis_index("s") * _NC + lax.axis_index("c")
    base_row = wid * _ROWS_PER_W
    idx_bufs = (idx0, idx1)
    rows_bufs = (rows0, rows1)
    sems = (sem0, sem1)

    def stage_and_fire(s, p):
        row0 = base_row + s * _CB
        pltpu.sync_copy(ids_hbm.at[pl.ds(row0 * _L, _IDX_PER_STEP)],
                        idx_bufs[p])
        for c in range(_NCHUNK):
            off = c * _CHUNK
            pltpu.async_copy(
                emb_hbm.at[idx_bufs[p].at[pl.ds(off, _CHUNK)]],
                rows_bufs[p].at[pl.ds(off, _CHUNK)],
                sems[p],
            )

    inv = jnp.float32(1.0 / _L)

    def process(s, p):
        pltpu.make_async_copy(
            emb_hbm.at[pl.ds(0, _IDX_PER_STEP)], rows_bufs[p], sems[p]
        ).wait()

        rows_v = rows_bufs[p]

        @plsc.parallel_loop(0, _CB)
        def _pool_one(i):
            j0 = i * _L

            def add_tok(l, acc):
                j = j0 + l
                return (acc[0] + rows_v[j, pl.ds(0, 16)],
                        acc[1] + rows_v[j, pl.ds(16, 16)])

            a0, a1 = lax.fori_loop(
                0, _L, add_tok,
                (jnp.zeros((16,), jnp.float32), jnp.zeros((16,), jnp.float32)),
                unroll=True,
            )
            acc_v[i, pl.ds(0, 16)] = a0 * inv
            acc_v[i, pl.ds(16, 16)] = a1 * inv

        pltpu.sync_copy(acc_v, out_hbm.at[pl.ds(base_row + s * _CB, _CB)])

    stage_and_fire(0, 0)

    def pstep(t, carry):
        s0, s1, s2 = 2 * t, 2 * t + 1, 2 * t + 2
        stage_and_fire(s1, 1)
        process(s0, 0)

        @pl.when(s2 < _NSTEP)
        def _():
            stage_and_fire(s2, 0)

        process(s1, 1)
        return carry

    lax.fori_loop(0, _NSTEP // 2, pstep, 0)


@jax.jit
def _pool(ids_flat, emb):
    mesh = plsc.VectorSubcoreMesh(core_axis_name="c", subcore_axis_name="s")
    return pl.kernel(
        _pool_body,
        out_type=jax.ShapeDtypeStruct((_B, _D), jnp.float32),
        mesh=mesh,
        scratch_types=[
            pltpu.VMEM((_IDX_PER_STEP,), jnp.int32),
            pltpu.VMEM((_IDX_PER_STEP,), jnp.int32),
            pltpu.VMEM((_IDX_PER_STEP, _D), jnp.float32),
            pltpu.VMEM((_IDX_PER_STEP, _D), jnp.float32),
            pltpu.VMEM((_CB, _D), jnp.float32),
            pltpu.SemaphoreType.DMA,
            pltpu.SemaphoreType.DMA,
        ],
        compiler_params=pltpu.CompilerParams(use_tc_tiling_on_sc=False),
    )(emb, ids_flat)


_RK = 7
_RCW = _RK * 128
_NFULL = _VOCAB // 128
_RNCHUNK = _NFULL // _RK
_TAIL_COL = _NFULL * 128
_TAIL_W = _VOCAB - _TAIL_COL


def _relayout_body(embt_hbm, tail_hbm, out_hbm,
                   sl0, sl1, ov0, ov1, sem0, sem1, semo0, semo1):
    wid = lax.axis_index("s") * _NC + lax.axis_index("c")
    lane = lax.iota(jnp.int32, 16)
    lane32 = lane * _D

    nchunk_w = jnp.where(wid < _RNCHUNK % _NW,
                         _RNCHUNK // _NW + 1, _RNCHUNK // _NW)

    def fire(j, sl, sem):
        col0 = pl.multiple_of((wid + j * _NW) * _RCW, 128)
        for jb in range(4):
            pltpu.async_copy(
                embt_hbm.at[pl.ds(jb * 8, 8), pl.ds(col0, _RCW)],
                sl.at[pl.ds(jb * 8, 8), pl.ds(0, _RCW)], sem)

    def process(j, t, sl, ov, sem, semo):
        pltpu.make_async_copy(
            embt_hbm.at[pl.ds(0, 32), pl.ds(0, _RCW)], sl, sem).wait()

        @pl.when(t > 0)
        def _():
            pltpu.make_async_copy(
                out_hbm.at[pl.ds(0, _RCW * _D)], ov, semo).wait()

        @plsc.parallel_loop(0, _RCW // 16)
        def _rblock(rb):
            r0 = rb * 16
            cvec = lane + r0
            sbase = r0 * _D
            for db in range(2):
                for g in range(0, 16, 4):
                    pairs = []
                    for k in range(g, g + 4):
                        perm = jnp.bitwise_and(lane + k, 15)
                        vals = plsc.load_gather(sl, [perm + db * 16, cvec])
                        pairs.append((lane32 + perm + (sbase + db * 16), vals))
                    for sidx, vals in pairs:
                        plsc.store_scatter(ov, [sidx], vals)

        col0 = pl.multiple_of((wid + j * _NW) * _RCW, 128)
        pltpu.async_copy(ov, out_hbm.at[pl.ds(col0 * _D, _RCW * _D)], semo)

    fire(0, sl0, sem0)
    nt2 = (nchunk_w + 1) // 2

    def step(t, carry):
        j0, j1, j2 = 2 * t, 2 * t + 1, 2 * t + 2

        @pl.when(j1 < nchunk_w)
        def _():
            fire(j1, sl1, sem1)

        process(j0, t, sl0, ov0, sem0, semo0)

        @pl.when(j2 < nchunk_w)
        def _():
            fire(j2, sl0, sem0)

        @pl.when(j1 < nchunk_w)
        def _():
            process(j1, t, sl1, ov1, sem1, semo1)

        return carry

    lax.fori_loop(0, nt2, step, 0)
    pltpu.make_async_copy(
        out_hbm.at[pl.ds(0, _RCW * _D)], ov0, semo0).wait()
    pltpu.make_async_copy(
        out_hbm.at[pl.ds(0, _RCW * _D)], ov1, semo1).wait()

    @pl.when(wid == _NW - 1)
    def _tail():
        pltpu.sync_copy(tail_hbm, ov0.at[pl.ds(0, _TAIL_W * _D)])
        pltpu.sync_copy(ov0.at[pl.ds(0, _TAIL_W * _D)],
                        out_hbm.at[pl.ds(_TAIL_COL * _D, _TAIL_W * _D)])


@jax.jit
def _relayout(emb):
    mesh = plsc.VectorSubcoreMesh(core_axis_name="c", subcore_axis_name="s")
    out_flat = pl.kernel(
        _relayout_body,
        out_type=jax.ShapeDtypeStruct((_VOCAB * _D,), jnp.float32),
        mesh=mesh,
        scratch_types=[
            pltpu.VMEM((32, _RCW), jnp.float32),
            pltpu.VMEM((32, _RCW), jnp.float32),
            pltpu.VMEM((_RCW * _D,), jnp.float32),
            pltpu.VMEM((_RCW * _D,), jnp.float32),
            pltpu.SemaphoreType.DMA,
            pltpu.SemaphoreType.DMA,
            pltpu.SemaphoreType.DMA,
            pltpu.SemaphoreType.DMA,
        ],
        compiler_params=pltpu.CompilerParams(use_tc_tiling_on_sc=True,
                                             needs_layout_passes=False,
                                             disable_bounds_checks=True),
    )(emb.T, emb[_TAIL_COL:].reshape(-1))
    return out_flat.reshape(_VOCAB, _D)


def _mlp_body(x_ref, w1_ref, b1_ref, w2_ref, b2_ref, o_ref):
    x = x_ref[...]
    h = jnp.dot(x, w1_ref[...], preferred_element_type=jnp.float32)
    h = jnp.maximum(h + b1_ref[...], 0.0)
    o_ref[...] = (
        jnp.dot(h, w2_ref[...], preferred_element_type=jnp.float32)
        + b2_ref[...]
    )


@jax.jit
def _mlp(x, W1, b1, W2, b2):
    blk = 2048
    grid = _B // blk
    return pl.pallas_call(
        _mlp_body,
        grid=(grid,),
        in_specs=[
            pl.BlockSpec((blk, _D), lambda i: (i, 0)),
            pl.BlockSpec((_D, _H), lambda i: (0, 0)),
            pl.BlockSpec((1, _H), lambda i: (0, 0)),
            pl.BlockSpec((_H, _C), lambda i: (0, 0)),
            pl.BlockSpec((1, _C), lambda i: (0, 0)),
        ],
        out_specs=pl.BlockSpec((blk, _C), lambda i: (i, 0)),
        out_shape=jax.ShapeDtypeStruct((_B, _C), jnp.float32),
    )(x, W1, b1.reshape(1, _H), W2, b2.reshape(1, _C))


def kernel(ids, emb, W1, b1, W2, b2):
    ids_flat = ids.reshape(-1).astype(jnp.int32)
    emb_lin = _relayout(emb)
    pooled = _pool(ids_flat, emb_lin)
    return _mlp(pooled, W1, b1, W2, b2)


# speedup vs baseline: 4.4529x; 1.0365x over previous
import functools

import jax
import jax.numpy as jnp
from jax import lax
from jax.experimental import pallas as pl
from jax.experimental.pallas import tpu as pltpu
from jax.experimental.pallas import tpu_sc as plsc

_VOCAB = 1000000
_D = 32
_H = 64
_C = 3
_B = 16384
_L = 50

_NC = 2
_NS = 16
_NW = _NC * _NS

_CB = 32
_CHUNK = 80
_IDX_PER_STEP = _CB * _L
_NCHUNK = _IDX_PER_STEP // _CHUNK
_ROWS_PER_W = _B // _NW
_NSTEP = _ROWS_PER_W // _CB


def _pool_body(emb_hbm, ids_hbm, out_hbm,
               idx0, idx1, rows0, rows1, acc_v, sem0, sem1):
    wid = lax.axis_index("s") * _NC + lax.axis_index("c")
    base_row = wid * _ROWS_PER_W
    idx_bufs = (idx0, idx1)
    rows_bufs = (rows0, rows1)
    sems = (sem0, sem1)

    def stage_and_fire(s, p):
        row0 = base_row + s * _CB
        pltpu.sync_copy(ids_hbm.at[pl.ds(row0 * _L, _IDX_PER_STEP)],
                        idx_bufs[p])
        for c in range(_NCHUNK):
            off = c * _CHUNK
            pltpu.async_copy(
                emb_hbm.at[idx_bufs[p].at[pl.ds(off, _CHUNK)]],
                rows_bufs[p].at[pl.ds(off, _CHUNK)],
                sems[p],
            )

    inv = jnp.float32(1.0 / _L)

    def process(s, p):
        pltpu.make_async_copy(
            emb_hbm.at[pl.ds(0, _IDX_PER_STEP)], rows_bufs[p], sems[p]
        ).wait()

        rows_v = rows_bufs[p]

        @plsc.parallel_loop(0, _CB)
        def _pool_one(i):
            j0 = i * _L

            def add_tok(l, acc):
                j = j0 + l
                return (acc[0] + rows_v[j, pl.ds(0, 16)],
                        acc[1] + rows_v[j, pl.ds(16, 16)])

            a0, a1 = lax.fori_loop(
                0, _L, add_tok,
                (jnp.zeros((16,), jnp.float32), jnp.zeros((16,), jnp.float32)),
                unroll=True,
            )
            acc_v[pl.ds(i * _D, 16)] = a0 * inv
            acc_v[pl.ds(i * _D + 16, 16)] = a1 * inv

        pltpu.sync_copy(
            acc_v, out_hbm.at[pl.ds((base_row + s * _CB) * _D, _CB * _D)])

    stage_and_fire(0, 0)

    def pstep(t, carry):
        s0, s1, s2 = 2 * t, 2 * t + 1, 2 * t + 2
        stage_and_fire(s1, 1)
        process(s0, 0)

        @pl.when(s2 < _NSTEP)
        def _():
            stage_and_fire(s2, 0)

        process(s1, 1)
        return carry

    lax.fori_loop(0, _NSTEP // 2, pstep, 0)


@jax.jit
def _pool(ids_flat, emb):
    mesh = plsc.VectorSubcoreMesh(core_axis_name="c", subcore_axis_name="s")
    return pl.kernel(
        _pool_body,
        out_type=jax.ShapeDtypeStruct((_B * _D,), jnp.float32),
        mesh=mesh,
        scratch_types=[
            pltpu.VMEM((_IDX_PER_STEP,), jnp.int32),
            pltpu.VMEM((_IDX_PER_STEP,), jnp.int32),
            pltpu.VMEM((_IDX_PER_STEP, _D), jnp.float32),
            pltpu.VMEM((_IDX_PER_STEP, _D), jnp.float32),
            pltpu.VMEM((_CB * _D,), jnp.float32),
            pltpu.SemaphoreType.DMA,
            pltpu.SemaphoreType.DMA,
        ],
        compiler_params=pltpu.CompilerParams(use_tc_tiling_on_sc=False),
    )(emb, ids_flat)


_RK = 7
_RCW = _RK * 128
_NFULL = _VOCAB // 128
_RNCHUNK = _NFULL // _RK
_TAIL_COL = _NFULL * 128
_TAIL_W = _VOCAB - _TAIL_COL


def _relayout_body(embt_hbm, tail_hbm, out_hbm,
                   sl0, sl1, ov0, ov1, sem0, sem1, semo0, semo1):
    wid = lax.axis_index("s") * _NC + lax.axis_index("c")
    lane = lax.iota(jnp.int32, 16)
    lane32 = lane * _D

    nchunk_w = jnp.where(wid < _RNCHUNK % _NW,
                         _RNCHUNK // _NW + 1, _RNCHUNK // _NW)

    def fire(j, sl, sem):
        col0 = pl.multiple_of((wid + j * _NW) * _RCW, 128)
        for jb in range(4):
            pltpu.async_copy(
                embt_hbm.at[pl.ds(jb * 8, 8), pl.ds(col0, _RCW)],
                sl.at[pl.ds(jb * 8, 8), pl.ds(0, _RCW)], sem)

    def process(j, t, sl, ov, sem, semo):
        pltpu.make_async_copy(
            embt_hbm.at[pl.ds(0, 32), pl.ds(0, _RCW)], sl, sem).wait()

        @pl.when(t > 0)
        def _():
            pltpu.make_async_copy(
                out_hbm.at[pl.ds(0, _RCW * _D)], ov, semo).wait()

        @plsc.parallel_loop(0, _RCW // 16)
        def _rblock(rb):
            r0 = rb * 16
            cvec = lane + r0
            sbase = r0 * _D
            for db in range(2):
                for g in range(0, 16, 4):
                    pairs = []
                    for k in range(g, g + 4):
                        perm = jnp.bitwise_and(lane + k, 15)
                        vals = plsc.load_gather(sl, [perm + db * 16, cvec])
                        pairs.append((lane32 + perm + (sbase + db * 16), vals))
                    for sidx, vals in pairs:
                        plsc.store_scatter(ov, [sidx], vals)

        col0 = pl.multiple_of((wid + j * _NW) * _RCW, 128)
        pltpu.async_copy(ov, out_hbm.at[pl.ds(col0 * _D, _RCW * _D)], semo)

    fire(0, sl0, sem0)
    nt2 = (nchunk_w + 1) // 2

    def step(t, carry):
        j0, j1, j2 = 2 * t, 2 * t + 1, 2 * t + 2

        @pl.when(j1 < nchunk_w)
        def _():
            fire(j1, sl1, sem1)

        process(j0, t, sl0, ov0, sem0, semo0)

        @pl.when(j2 < nchunk_w)
        def _():
            fire(j2, sl0, sem0)

        @pl.when(j1 < nchunk_w)
        def _():
            process(j1, t, sl1, ov1, sem1, semo1)

        return carry

    lax.fori_loop(0, nt2, step, 0)
    pltpu.make_async_copy(
        out_hbm.at[pl.ds(0, _RCW * _D)], ov0, semo0).wait()
    pltpu.make_async_copy(
        out_hbm.at[pl.ds(0, _RCW * _D)], ov1, semo1).wait()

    @pl.when(wid == _NW - 1)
    def _tail():
        pltpu.sync_copy(tail_hbm, ov0.at[pl.ds(0, _TAIL_W * _D)])
        pltpu.sync_copy(ov0.at[pl.ds(0, _TAIL_W * _D)],
                        out_hbm.at[pl.ds(_TAIL_COL * _D, _TAIL_W * _D)])


@jax.jit
def _relayout(emb):
    mesh = plsc.VectorSubcoreMesh(core_axis_name="c", subcore_axis_name="s")
    out_flat = pl.kernel(
        _relayout_body,
        out_type=jax.ShapeDtypeStruct((_VOCAB * _D,), jnp.float32),
        mesh=mesh,
        scratch_types=[
            pltpu.VMEM((32, _RCW), jnp.float32),
            pltpu.VMEM((32, _RCW), jnp.float32),
            pltpu.VMEM((_RCW * _D,), jnp.float32),
            pltpu.VMEM((_RCW * _D,), jnp.float32),
            pltpu.SemaphoreType.DMA,
            pltpu.SemaphoreType.DMA,
            pltpu.SemaphoreType.DMA,
            pltpu.SemaphoreType.DMA,
        ],
        compiler_params=pltpu.CompilerParams(use_tc_tiling_on_sc=True,
                                             needs_layout_passes=False,
                                             disable_bounds_checks=True),
    )(emb.T, emb[_TAIL_COL:].reshape(-1))
    return out_flat.reshape(_VOCAB, _D)


def _mlp_body(x_ref, w1_ref, b1_ref, w2_ref, b2_ref, o_ref):
    x = x_ref[...]
    h = jnp.dot(x, w1_ref[...], preferred_element_type=jnp.float32)
    h = jnp.maximum(h + b1_ref[...], 0.0)
    o_ref[...] = (
        jnp.dot(h, w2_ref[...], preferred_element_type=jnp.float32)
        + b2_ref[...]
    )


@jax.jit
def _mlp(x4, W1, b1, W2, b2):
    rows = _B // 4
    blk = rows // 2
    w1b = jax.scipy.linalg.block_diag(W1, W1, W1, W1)
    b1b = jnp.tile(b1, 4).reshape(1, 4 * _H)
    w2b = jax.scipy.linalg.block_diag(W2, W2, W2, W2)
    b2b = jnp.tile(b2, 4).reshape(1, 4 * _C)
    out4 = pl.pallas_call(
        _mlp_body,
        grid=(2,),
        in_specs=[
            pl.BlockSpec((blk, 128), lambda i: (i, 0)),
            pl.BlockSpec((128, 4 * _H), lambda i: (0, 0)),
            pl.BlockSpec((1, 4 * _H), lambda i: (0, 0)),
            pl.BlockSpec((4 * _H, 4 * _C), lambda i: (0, 0)),
            pl.BlockSpec((1, 4 * _C), lambda i: (0, 0)),
        ],
        out_specs=pl.BlockSpec((blk, 4 * _C), lambda i: (i, 0)),
        out_shape=jax.ShapeDtypeStruct((rows, 4 * _C), jnp.float32),
    )(x4, w1b, b1b, w2b, b2b)
    return out4.reshape(_B, _C)


def kernel(ids, emb, W1, b1, W2, b2):
    ids_flat = ids.reshape(-1).astype(jnp.int32)
    emb_lin = _relayout(emb)
    pooled_flat = _pool(ids_flat, emb_lin)
    return _mlp(pooled_flat.reshape(_B // 4, 128), W1, b1, W2, b2)
